# baseline (speedup 1.0000x reference)
import jax
import jax.numpy as jnp
from jax import lax
from jax.experimental import pallas as pl
from jax.experimental.pallas import tpu as pltpu
from jax.experimental.pallas import tpu_sc as plsc

N_NODES = 10000
N_EXT = 10240
E_EDGES = 320000
D = 128
CK = 32
GRP = 16
G_TILE = 40
TOTG = 16 * G_TILE
E_PAD = TOTG * GRP * CK
BN = 2048
SENTINEL = -1e6


def _linear_body(x_ref, w_ref, b_ref, a_ref, batt_ref, h_ref, s_ref):
    i = pl.program_id(0)
    h = jnp.dot(x_ref[...], w_ref[...], preferred_element_type=jnp.float32)
    h = h + b_ref[...]
    h_ref[...] = h
    s12 = lax.dot_general(a_ref[...], h, (((1,), (1,)), ((), ())),
                          preferred_element_type=jnp.float32)
    is_s1 = lax.broadcasted_iota(jnp.int32, (2, 1), 0) == 0
    s12 = s12 + jnp.where(is_s1, batt_ref[0, 0], jnp.float32(0.0))
    rowid = i * BN + lax.broadcasted_iota(jnp.int32, (1, BN), 1)
    s_ref[...] = jnp.where(rowid >= N_NODES, jnp.float32(SENTINEL), s12)


def _sc_body(h_hbm, s_hbm, rows_hbm, cols_hbm, w_hbm,
             outp_hbm, outz_hbm,
             s1_v, s2_v, rows_v, cols_v, w_v, c0_v, c1_v, gbuf, gbuf_b,
             zacc, acc, sem0, sem1):
    sid = lax.axis_index("s")
    pltpu.sync_copy(s_hbm.at[0], s1_v)
    pltpu.sync_copy(s_hbm.at[1], s2_v)
    rows_per_sub = N_EXT // 16

    def zrow(r, c):
        for u in range(D // 16):
            gbuf[0, r, pl.ds(u * 16, 16)] = jnp.zeros((16,), jnp.float32)
        return c

    lax.fori_loop(0, CK, zrow, 0)
    for b in range(rows_per_sub // CK):
        pltpu.sync_copy(gbuf.at[0],
                        acc.at[pl.ds(sid * rows_per_sub + b * CK, CK)])
    zacc[...] = jnp.zeros((16,), jnp.float32)
    plsc.subcore_barrier()

    base_g = sid * G_TILE
    bodies_per_group = GRP // 2

    def compute_c(pp, k, c_ref):
        for k4 in range(CK // 16):
            sl = pl.ds(k4 * 16, 16)
            ridx = rows_v[pp, k, sl]
            cidx = cols_v[pp, k, sl]
            t = plsc.load_gather(s1_v, [ridx]) + plsc.load_gather(s2_v, [cidx])
            t = jnp.where(t >= 0.0, t, 0.01 * t)
            e = jnp.exp(t)
            zacc[...] = zacc[...] + e
            c_ref[sl] = e * w_v[pp, k, sl]

    def scale(slot, c_ref):
        def grp16(q, c2):
            cvec = c_ref[pl.ds(q * 16, 16)]
            for i in range(16):
                r = q * 16 + i
                cs = cvec[i]
                for u in range(4):
                    slq = pl.ds(u * 16, 16)
                    v = gbuf_b[slot, r, slq]
                    lo = plsc.bitcast(v << 16, jnp.float32)
                    hi = plsc.bitcast(v & jnp.int32(-65536), jnp.float32)
                    gbuf[slot, r, slq] = lo * cs
                    gbuf[slot, r, pl.ds(64 + u * 16, 16)] = hi * cs
            return c2

        lax.fori_loop(0, CK // 16, grp16, 0)

    def body(t, carry):
        lg = t // bodies_per_group
        gg = base_g + lg
        pp = lax.rem(lg, 2)
        k0 = lax.rem(2 * t, GRP)
        k1 = k0 + 1

        @pl.when(lax.rem(t, bodies_per_group) == 0)
        def _stage():
            pltpu.sync_copy(rows_hbm.at[gg], rows_v.at[pp])
            pltpu.sync_copy(cols_hbm.at[gg], cols_v.at[pp])
            pltpu.sync_copy(w_hbm.at[gg], w_v.at[pp])

        compute_c(pp, k0, c0_v)

        @pl.when(t > 0)
        def _drain0():
            pltpu.make_async_copy(
                gbuf.at[0], acc.at[rows_v.at[pp, k0]], sem0).wait()

        cpg0 = pltpu.async_copy(h_hbm.at[cols_v.at[pp, k0]], gbuf_b.at[0],
                                sem0)

        compute_c(pp, k1, c1_v)

        @pl.when(t > 0)
        def _drain1():
            pltpu.make_async_copy(
                gbuf.at[1], acc.at[rows_v.at[pp, k1]], sem1).wait()

        cpg1 = pltpu.async_copy(h_hbm.at[cols_v.at[pp, k1]], gbuf_b.at[1],
                                sem1)

        cpg0.wait()
        scale(0, c0_v)
        pltpu.async_copy(gbuf.at[0], acc.at[rows_v.at[pp, k0]], sem0,
                         add=True)

        cpg1.wait()
        scale(1, c1_v)
        pltpu.async_copy(gbuf.at[1], acc.at[rows_v.at[pp, k1]], sem1,
                         add=True)
        return carry

    lax.fori_loop(0, G_TILE * bodies_per_group, body, 0)
    pltpu.make_async_copy(gbuf.at[0], acc.at[rows_v.at[0, 0]], sem0).wait()
    pltpu.make_async_copy(gbuf.at[1], acc.at[rows_v.at[0, 1]], sem1).wait()

    pltpu.sync_copy(zacc, outz_hbm.at[sid])
    plsc.subcore_barrier()
    pltpu.sync_copy(acc.at[pl.ds(sid * rows_per_sub, rows_per_sub)],
                    outp_hbm.at[pl.ds(sid * rows_per_sub, rows_per_sub)])


def _combine_body(h_ref, p_ref, z_ref, o_ref):
    inv = 1.0 / jnp.sum(z_ref[...])
    o_ref[...] = jnp.maximum(h_ref[...] + p_ref[...] * inv, 0.0)


def kernel(x, edge_index, edge_weight, W_lin, b_lin, W_att, b_att):
    f32 = jnp.float32
    x_ext = jnp.concatenate(
        [x.astype(f32), jnp.zeros((N_EXT - N_NODES, D), f32)], axis=0)
    a_r = W_att[:, 0].reshape(2, D).astype(f32)

    h, s = pl.pallas_call(
        _linear_body,
        grid=(N_EXT // BN,),
        in_specs=[
            pl.BlockSpec((BN, D), lambda i: (i, 0)),
            pl.BlockSpec((D, D), lambda i: (0, 0)),
            pl.BlockSpec((1, D), lambda i: (0, 0)),
            pl.BlockSpec((2, D), lambda i: (0, 0)),
            pl.BlockSpec((1, 1), lambda i: (0, 0)),
        ],
        out_specs=[
            pl.BlockSpec((BN, D), lambda i: (i, 0)),
            pl.BlockSpec((2, BN), lambda i: (0, i)),
        ],
        out_shape=[
            jax.ShapeDtypeStruct((N_EXT, D), f32),
            jax.ShapeDtypeStruct((2, N_EXT), f32),
        ],
    )(x_ext, W_lin.astype(f32), b_lin.reshape(1, D).astype(f32), a_r,
      b_att.reshape(1, 1).astype(f32))

    pad = E_PAD - E_EDGES
    rows_r = jnp.concatenate(
        [edge_index[0].astype(jnp.int32), jnp.zeros((pad,), jnp.int32)]
    ).reshape(TOTG, GRP, CK)
    cols_r = jnp.concatenate(
        [edge_index[1].astype(jnp.int32),
         jnp.full((pad,), N_NODES, jnp.int32)]
    ).reshape(TOTG, GRP, CK)
    w_r = jnp.concatenate(
        [edge_weight.astype(f32), jnp.zeros((pad,), f32)]
    ).reshape(TOTG, GRP, CK)
    h_pairs = jnp.stack([h[:, : D // 2], h[:, D // 2:]], axis=-1)
    h_bi = lax.bitcast_convert_type(h_pairs.astype(jnp.bfloat16), jnp.int32)

    mesh = plsc.VectorSubcoreMesh(core_axis_name="c", subcore_axis_name="s",
                                  num_cores=1)
    partials, zarr = pl.kernel(
        _sc_body,
        out_type=[
            jax.ShapeDtypeStruct((N_EXT, D), f32),
            jax.ShapeDtypeStruct((16, 16), f32),
        ],
        mesh=mesh,
        compiler_params=pltpu.CompilerParams(needs_layout_passes=False,
                                             use_tc_tiling_on_sc=False),
        scratch_types=[
            pltpu.VMEM((N_EXT,), f32),
            pltpu.VMEM((N_EXT,), f32),
            pltpu.VMEM((2, GRP, CK), jnp.int32),
            pltpu.VMEM((2, GRP, CK), jnp.int32),
            pltpu.VMEM((2, GRP, CK), f32),
            pltpu.VMEM((CK,), f32),
            pltpu.VMEM((CK,), f32),
            pltpu.VMEM((2, CK, D), f32),
            pltpu.VMEM((2, CK, D // 2), jnp.int32),
            pltpu.VMEM((16,), f32),
            pltpu.VMEM_SHARED((N_EXT, D), f32),
            pltpu.SemaphoreType.DMA,
            pltpu.SemaphoreType.DMA,
        ],
    )(h_bi, s, rows_r, cols_r, w_r)

    out = pl.pallas_call(
        _combine_body,
        grid=(N_EXT // BN,),
        in_specs=[
            pl.BlockSpec((BN, D), lambda i: (i, 0)),
            pl.BlockSpec((BN, D), lambda i: (i, 0)),
            pl.BlockSpec((16, 16), lambda i: (0, 0)),
        ],
        out_specs=pl.BlockSpec((BN, D), lambda i: (i, 0)),
        out_shape=jax.ShapeDtypeStruct((N_EXT, D), f32),
    )(h, partials, zarr)

    return out[:N_NODES]


# speedup vs baseline: 1.0268x; 1.0268x over previous
import jax
import jax.numpy as jnp
from jax import lax
from jax.experimental import pallas as pl
from jax.experimental.pallas import tpu as pltpu
from jax.experimental.pallas import tpu_sc as plsc

N_NODES = 10000
N_EXT = 10240
E_EDGES = 320000
D = 128
CK = 32
GRP = 16
G_TILE = 40
TOTG = 16 * G_TILE
E_PAD = TOTG * GRP * CK
BN = 2048
SENTINEL = -1e6


def _linear_body(x_ref, w_ref, b_ref, a_ref, batt_ref, h_ref, s_ref):
    i = pl.program_id(0)
    h = jnp.dot(x_ref[...], w_ref[...], preferred_element_type=jnp.float32)
    h = h + b_ref[...]
    h_ref[...] = h
    s12 = lax.dot_general(a_ref[...], h, (((1,), (1,)), ((), ())),
                          preferred_element_type=jnp.float32)
    is_s1 = lax.broadcasted_iota(jnp.int32, (2, 1), 0) == 0
    s12 = s12 + jnp.where(is_s1, batt_ref[0, 0], jnp.float32(0.0))
    rowid = i * BN + lax.broadcasted_iota(jnp.int32, (1, BN), 1)
    s_ref[...] = jnp.where(rowid >= N_NODES, jnp.float32(SENTINEL), s12)


def _sc_body(h_hbm, s_hbm, rows_hbm, cols_hbm, w_hbm,
             outp_hbm, outz_hbm,
             s1_v, s2_v, rows_v, cols_v, w_v, c0_v, c1_v, gbuf, gbuf_b,
             zacc, acc, sem0, sem1):
    cid = lax.axis_index("c")
    sid = lax.axis_index("s")
    wid = cid * 16 + sid
    on_fast = cid == 0
    zacc[...] = jnp.zeros((16,), jnp.float32)
    rows_per_sub = N_EXT // 16

    @pl.when(on_fast)
    def _setup():
        pltpu.sync_copy(s_hbm.at[0], s1_v)
        pltpu.sync_copy(s_hbm.at[1], s2_v)

        def zrow(r, c):
            for u in range(D // 16):
                gbuf[0, r, pl.ds(u * 16, 16)] = jnp.zeros((16,), jnp.float32)
            return c

        lax.fori_loop(0, CK, zrow, 0)
        for b in range(rows_per_sub // CK):
            pltpu.sync_copy(gbuf.at[0],
                            acc.at[pl.ds(sid * rows_per_sub + b * CK, CK)])

    plsc.subcore_barrier()

    base_g = sid * G_TILE
    bodies_per_group = GRP // 2

    def compute_c(pp, k, c_ref):
        for k4 in range(CK // 16):
            sl = pl.ds(k4 * 16, 16)
            ridx = rows_v[pp, k, sl]
            cidx = cols_v[pp, k, sl]
            t = plsc.load_gather(s1_v, [ridx]) + plsc.load_gather(s2_v, [cidx])
            t = jnp.where(t >= 0.0, t, 0.01 * t)
            e = jnp.exp(t)
            zacc[...] = zacc[...] + e
            c_ref[sl] = e * w_v[pp, k, sl]

    def scale(slot, c_ref):
        def grp16(q, c2):
            cvec = c_ref[pl.ds(q * 16, 16)]
            for i in range(16):
                r = q * 16 + i
                cs = cvec[i]
                for u in range(4):
                    slq = pl.ds(u * 16, 16)
                    v = gbuf_b[slot, r, slq]
                    lo = plsc.bitcast(v << 16, jnp.float32)
                    hi = plsc.bitcast(v & jnp.int32(-65536), jnp.float32)
                    gbuf[slot, r, slq] = lo * cs
                    gbuf[slot, r, pl.ds(64 + u * 16, 16)] = hi * cs
            return c2

        lax.fori_loop(0, CK // 16, grp16, 0)

    def body(t, carry):
        lg = t // bodies_per_group
        gg = base_g + lg
        pp = lax.rem(lg, 2)
        k0 = lax.rem(2 * t, GRP)
        k1 = k0 + 1

        @pl.when(lax.rem(t, bodies_per_group) == 0)
        def _stage():
            pltpu.sync_copy(rows_hbm.at[gg], rows_v.at[pp])
            pltpu.sync_copy(cols_hbm.at[gg], cols_v.at[pp])
            pltpu.sync_copy(w_hbm.at[gg], w_v.at[pp])

        compute_c(pp, k0, c0_v)

        @pl.when(t > 0)
        def _drain0():
            pltpu.make_async_copy(
                gbuf.at[0], acc.at[rows_v.at[pp, k0]], sem0).wait()

        cpg0 = pltpu.async_copy(h_hbm.at[cols_v.at[pp, k0]], gbuf_b.at[0],
                                sem0)

        compute_c(pp, k1, c1_v)

        @pl.when(t > 0)
        def _drain1():
            pltpu.make_async_copy(
                gbuf.at[1], acc.at[rows_v.at[pp, k1]], sem1).wait()

        cpg1 = pltpu.async_copy(h_hbm.at[cols_v.at[pp, k1]], gbuf_b.at[1],
                                sem1)

        cpg0.wait()
        scale(0, c0_v)
        pltpu.async_copy(gbuf.at[0], acc.at[rows_v.at[pp, k0]], sem0,
                         add=True)

        cpg1.wait()
        scale(1, c1_v)
        pltpu.async_copy(gbuf.at[1], acc.at[rows_v.at[pp, k1]], sem1,
                         add=True)
        return carry

    @pl.when(on_fast)
    def _main():
        lax.fori_loop(0, G_TILE * bodies_per_group, body, 0)
        pltpu.make_async_copy(gbuf.at[0], acc.at[rows_v.at[0, 0]],
                              sem0).wait()
        pltpu.make_async_copy(gbuf.at[1], acc.at[rows_v.at[0, 1]],
                              sem1).wait()

    pltpu.sync_copy(zacc, outz_hbm.at[wid])
    plsc.subcore_barrier()

    @pl.when(on_fast)
    def _copy_out():
        pltpu.sync_copy(acc.at[pl.ds(sid * rows_per_sub, rows_per_sub)],
                        outp_hbm.at[pl.ds(sid * rows_per_sub, rows_per_sub)])


def _combine_body(h_ref, p_ref, z_ref, o_ref):
    inv = 1.0 / jnp.sum(z_ref[...])
    o_ref[...] = jnp.maximum(h_ref[...] + p_ref[...] * inv, 0.0)


def kernel(x, edge_index, edge_weight, W_lin, b_lin, W_att, b_att):
    f32 = jnp.float32
    x_ext = jnp.concatenate(
        [x.astype(f32), jnp.zeros((N_EXT - N_NODES, D), f32)], axis=0)
    a_r = W_att[:, 0].reshape(2, D).astype(f32)

    h, s = pl.pallas_call(
        _linear_body,
        grid=(N_EXT // BN,),
        in_specs=[
            pl.BlockSpec((BN, D), lambda i: (i, 0)),
            pl.BlockSpec((D, D), lambda i: (0, 0)),
            pl.BlockSpec((1, D), lambda i: (0, 0)),
            pl.BlockSpec((2, D), lambda i: (0, 0)),
            pl.BlockSpec((1, 1), lambda i: (0, 0)),
        ],
        out_specs=[
            pl.BlockSpec((BN, D), lambda i: (i, 0)),
            pl.BlockSpec((2, BN), lambda i: (0, i)),
        ],
        out_shape=[
            jax.ShapeDtypeStruct((N_EXT, D), f32),
            jax.ShapeDtypeStruct((2, N_EXT), f32),
        ],
    )(x_ext, W_lin.astype(f32), b_lin.reshape(1, D).astype(f32), a_r,
      b_att.reshape(1, 1).astype(f32))

    pad = E_PAD - E_EDGES
    rows_r = jnp.concatenate(
        [edge_index[0].astype(jnp.int32), jnp.zeros((pad,), jnp.int32)]
    ).reshape(TOTG, GRP, CK)
    cols_r = jnp.concatenate(
        [edge_index[1].astype(jnp.int32),
         jnp.full((pad,), N_NODES, jnp.int32)]
    ).reshape(TOTG, GRP, CK)
    w_r = jnp.concatenate(
        [edge_weight.astype(f32), jnp.zeros((pad,), f32)]
    ).reshape(TOTG, GRP, CK)
    h_pairs = jnp.stack([h[:, : D // 2], h[:, D // 2:]], axis=-1)
    h_bi = lax.bitcast_convert_type(h_pairs.astype(jnp.bfloat16), jnp.int32)

    mesh = plsc.VectorSubcoreMesh(core_axis_name="c", subcore_axis_name="s")
    partials, zarr = pl.kernel(
        _sc_body,
        out_type=[
            jax.ShapeDtypeStruct((N_EXT, D), f32),
            jax.ShapeDtypeStruct((32, 16), f32),
        ],
        mesh=mesh,
        compiler_params=pltpu.CompilerParams(needs_layout_passes=False,
                                             use_tc_tiling_on_sc=False),
        scratch_types=[
            pltpu.VMEM((N_EXT,), f32),
            pltpu.VMEM((N_EXT,), f32),
            pltpu.VMEM((2, GRP, CK), jnp.int32),
            pltpu.VMEM((2, GRP, CK), jnp.int32),
            pltpu.VMEM((2, GRP, CK), f32),
            pltpu.VMEM((CK,), f32),
            pltpu.VMEM((CK,), f32),
            pltpu.VMEM((2, CK, D), f32),
            pltpu.VMEM((2, CK, D // 2), jnp.int32),
            pltpu.VMEM((16,), f32),
            pltpu.VMEM_SHARED((N_EXT, D), f32),
            pltpu.SemaphoreType.DMA,
            pltpu.SemaphoreType.DMA,
        ],
    )(h_bi, s, rows_r, cols_r, w_r)

    out = pl.pallas_call(
        _combine_body,
        grid=(N_EXT // BN,),
        in_specs=[
            pl.BlockSpec((BN, D), lambda i: (i, 0)),
            pl.BlockSpec((BN, D), lambda i: (i, 0)),
            pl.BlockSpec((32, 16), lambda i: (0, 0)),
        ],
        out_specs=pl.BlockSpec((BN, D), lambda i: (i, 0)),
        out_shape=jax.ShapeDtypeStruct((N_EXT, D), f32),
    )(h, partials, zarr)

    return out[:N_NODES]


# speedup vs baseline: 2.0234x; 1.9706x over previous
import jax
import jax.numpy as jnp
from jax import lax
from jax.experimental import pallas as pl
from jax.experimental.pallas import tpu as pltpu
from jax.experimental.pallas import tpu_sc as plsc

N_NODES = 10000
N_EXT = 10240
E_EDGES = 320000
D = 128
CK = 64
GRP = 8
G0 = 34
G1 = 6
TOTG = 16 * (G0 + G1)
E_PAD = TOTG * GRP * CK
BN = 2048
SENTINEL = -1e6


def _linear_body(x_ref, w_ref, b_ref, a_ref, batt_ref, h_ref, s_ref):
    i = pl.program_id(0)
    h = jnp.dot(x_ref[...], w_ref[...], preferred_element_type=jnp.float32)
    h = h + b_ref[...]
    h_ref[...] = h
    s12 = lax.dot_general(a_ref[...], h, (((1,), (1,)), ((), ())),
                          preferred_element_type=jnp.float32)
    is_s1 = lax.broadcasted_iota(jnp.int32, (2, 1), 0) == 0
    s12 = s12 + jnp.where(is_s1, batt_ref[0, 0], jnp.float32(0.0))
    rowid = i * BN + lax.broadcasted_iota(jnp.int32, (1, BN), 1)
    s_ref[...] = jnp.where(rowid >= N_NODES, jnp.float32(SENTINEL), s12)


def _sc_body(h_hbm, s_hbm, rows_hbm, cols_hbm, w_hbm,
             outp_hbm, outz_hbm,
             s1_v, s2_v, rows_v, cols_v, w_v, c0_v, c1_v, gbuf, zacc, acc,
             sem0, sem1):
    cid = lax.axis_index("c")
    sid = lax.axis_index("s")
    wid = cid * 16 + sid
    pltpu.sync_copy(s_hbm.at[0], s1_v)
    pltpu.sync_copy(s_hbm.at[1], s2_v)
    rows_per_sub = N_EXT // 16

    def zrow(r, c):
        for u in range(D // 16):
            gbuf[0, r, pl.ds(u * 16, 16)] = jnp.zeros((16,), jnp.float32)
        return c

    lax.fori_loop(0, CK, zrow, 0)
    for b in range(rows_per_sub // CK):
        pltpu.sync_copy(gbuf.at[0],
                        acc.at[pl.ds(sid * rows_per_sub + b * CK, CK)])
    zacc[...] = jnp.zeros((16,), jnp.float32)
    plsc.subcore_barrier()

    n_groups = jnp.where(cid == 0, G0, G1)
    base_g = cid * (16 * G0) + sid * n_groups
    bodies_per_group = GRP // 2

    def compute_c(pp, k, c_ref):
        for k4 in range(CK // 16):
            sl = pl.ds(k4 * 16, 16)
            ridx = rows_v[pp, k, sl]
            cidx = cols_v[pp, k, sl]
            t = plsc.load_gather(s1_v, [ridx]) + plsc.load_gather(s2_v, [cidx])
            t = jnp.where(t >= 0.0, t, 0.01 * t)
            e = jnp.exp(t)
            zacc[...] = zacc[...] + e
            c_ref[sl] = e * w_v[pp, k, sl]

    def scale(slot, c_ref):
        def grp16(q, c2):
            cvec = c_ref[pl.ds(q * 16, 16)]
            for i in range(16):
                r = q * 16 + i
                cs = cvec[i]
                for u in range(D // 16):
                    slq = pl.ds(u * 16, 16)
                    gbuf[slot, r, slq] = gbuf[slot, r, slq] * cs
            return c2

        lax.fori_loop(0, CK // 16, grp16, 0)

    def body(t, carry):
        lg = t // bodies_per_group
        gg = base_g + lg
        pp = lax.rem(lg, 2)
        k0 = lax.rem(2 * t, GRP)
        k1 = k0 + 1

        @pl.when(lax.rem(t, bodies_per_group) == 0)
        def _stage():
            pltpu.sync_copy(rows_hbm.at[gg], rows_v.at[pp])
            pltpu.sync_copy(cols_hbm.at[gg], cols_v.at[pp])
            pltpu.sync_copy(w_hbm.at[gg], w_v.at[pp])

        compute_c(pp, k0, c0_v)

        @pl.when(t > 0)
        def _drain0():
            pltpu.make_async_copy(
                gbuf.at[0], acc.at[rows_v.at[pp, k0]], sem0).wait()

        cpg0 = pltpu.async_copy(h_hbm.at[cols_v.at[pp, k0]], gbuf.at[0], sem0)

        compute_c(pp, k1, c1_v)

        @pl.when(t > 0)
        def _drain1():
            pltpu.make_async_copy(
                gbuf.at[1], acc.at[rows_v.at[pp, k1]], sem1).wait()

        cpg1 = pltpu.async_copy(h_hbm.at[cols_v.at[pp, k1]], gbuf.at[1], sem1)

        cpg0.wait()
        scale(0, c0_v)
        pltpu.async_copy(gbuf.at[0], acc.at[rows_v.at[pp, k0]], sem0,
                         add=True)

        cpg1.wait()
        scale(1, c1_v)
        pltpu.async_copy(gbuf.at[1], acc.at[rows_v.at[pp, k1]], sem1,
                         add=True)
        return carry

    lax.fori_loop(0, n_groups * bodies_per_group, body, 0)
    pltpu.make_async_copy(gbuf.at[0], acc.at[rows_v.at[0, 0]], sem0).wait()
    pltpu.make_async_copy(gbuf.at[1], acc.at[rows_v.at[0, 1]], sem1).wait()

    pltpu.sync_copy(zacc, outz_hbm.at[wid])
    plsc.subcore_barrier()
    pltpu.sync_copy(acc.at[pl.ds(sid * rows_per_sub, rows_per_sub)],
                    outp_hbm.at[cid, pl.ds(sid * rows_per_sub, rows_per_sub)])


def _combine_body(h_ref, p_ref, z_ref, o_ref):
    inv = 1.0 / jnp.sum(z_ref[...])
    o_ref[...] = jnp.maximum(h_ref[...] + (p_ref[0] + p_ref[1]) * inv, 0.0)


def kernel(x, edge_index, edge_weight, W_lin, b_lin, W_att, b_att):
    f32 = jnp.float32
    x_ext = jnp.concatenate(
        [x.astype(f32), jnp.zeros((N_EXT - N_NODES, D), f32)], axis=0)
    a_r = W_att[:, 0].reshape(2, D).astype(f32)

    h, s = pl.pallas_call(
        _linear_body,
        grid=(N_EXT // BN,),
        in_specs=[
            pl.BlockSpec((BN, D), lambda i: (i, 0)),
            pl.BlockSpec((D, D), lambda i: (0, 0)),
            pl.BlockSpec((1, D), lambda i: (0, 0)),
            pl.BlockSpec((2, D), lambda i: (0, 0)),
            pl.BlockSpec((1, 1), lambda i: (0, 0)),
        ],
        out_specs=[
            pl.BlockSpec((BN, D), lambda i: (i, 0)),
            pl.BlockSpec((2, BN), lambda i: (0, i)),
        ],
        out_shape=[
            jax.ShapeDtypeStruct((N_EXT, D), f32),
            jax.ShapeDtypeStruct((2, N_EXT), f32),
        ],
    )(x_ext, W_lin.astype(f32), b_lin.reshape(1, D).astype(f32), a_r,
      b_att.reshape(1, 1).astype(f32))

    pad = E_PAD - E_EDGES
    rows_r = jnp.concatenate(
        [edge_index[0].astype(jnp.int32), jnp.zeros((pad,), jnp.int32)]
    ).reshape(TOTG, GRP, CK)
    cols_r = jnp.concatenate(
        [edge_index[1].astype(jnp.int32),
         jnp.full((pad,), N_NODES, jnp.int32)]
    ).reshape(TOTG, GRP, CK)
    w_r = jnp.concatenate(
        [edge_weight.astype(f32), jnp.zeros((pad,), f32)]
    ).reshape(TOTG, GRP, CK)

    mesh = plsc.VectorSubcoreMesh(core_axis_name="c", subcore_axis_name="s")
    partials, zarr = pl.kernel(
        _sc_body,
        out_type=[
            jax.ShapeDtypeStruct((2, N_EXT, D), f32),
            jax.ShapeDtypeStruct((32, 16), f32),
        ],
        mesh=mesh,
        compiler_params=pltpu.CompilerParams(needs_layout_passes=False),
        scratch_types=[
            pltpu.VMEM((N_EXT,), f32),
            pltpu.VMEM((N_EXT,), f32),
            pltpu.VMEM((2, GRP, CK), jnp.int32),
            pltpu.VMEM((2, GRP, CK), jnp.int32),
            pltpu.VMEM((2, GRP, CK), f32),
            pltpu.VMEM((CK,), f32),
            pltpu.VMEM((CK,), f32),
            pltpu.VMEM((2, CK, D), f32),
            pltpu.VMEM((16,), f32),
            pltpu.VMEM_SHARED((N_EXT, D), f32),
            pltpu.SemaphoreType.DMA,
            pltpu.SemaphoreType.DMA,
        ],
    )(h, s, rows_r, cols_r, w_r)

    out = pl.pallas_call(
        _combine_body,
        grid=(N_EXT // BN,),
        in_specs=[
            pl.BlockSpec((BN, D), lambda i: (i, 0)),
            pl.BlockSpec((2, BN, D), lambda i: (0, i, 0)),
            pl.BlockSpec((32, 16), lambda i: (0, 0)),
        ],
        out_specs=pl.BlockSpec((BN, D), lambda i: (i, 0)),
        out_shape=jax.ShapeDtypeStruct((N_EXT, D), f32),
    )(h, partials, zarr)

    return out[:N_NODES]
